# baseline (speedup 1.0000x reference)
import functools

import jax
import jax.numpy as jnp
from jax import lax
from jax.experimental import pallas as pl
from jax.experimental.pallas import tpu as pltpu
from jax.experimental.pallas import tpu_sc as plsc

_N = 10000
_E = 320000
_D_IN = 128
_H = 64
_G = 64
_EPS = 1e-5

_NC = 2
_NS = 16
_NW = _NC * _NS
_CH = 128
_NCHUNK = 80
_EPW = _CH * _NCHUNK
_EPAD = _NW * _EPW
_NP = 10240
_RPT = _NP // _NS

_BM = 2000
_NB = _N // _BM


def _interleave_perm_matrix():
    import numpy as _np
    perm = _np.zeros((_H,), _np.int64)
    for blk in range(_H // 32):
        for i in range(16):
            perm[blk * 32 + 2 * i] = blk * 32 + i
            perm[blk * 32 + 2 * i + 1] = blk * 32 + 16 + i
    m = _np.zeros((_H, _H), _np.float32)
    for col in range(_H):
        m[perm[col], col] = 1.0
    return m


_PY = _interleave_perm_matrix()

_sc_mesh = plsc.VectorSubcoreMesh(
    core_axis_name="c", subcore_axis_name="s", num_cores=_NC, num_subcores=_NS)


@functools.partial(
    pl.kernel,
    out_type=jax.ShapeDtypeStruct((_NC * 80, 128), jnp.float32),
    mesh=_sc_mesh,
    compiler_params=pltpu.CompilerParams(needs_layout_passes=False, use_tc_tiling_on_sc=False),
    scratch_types=[
        pltpu.VMEM((_NCHUNK, _CH), jnp.int32),
        pltpu.VMEM((_NCHUNK, _CH), jnp.float32),
        pltpu.VMEM((80, 128), jnp.float32),
        pltpu.VMEM((16, 128), jnp.float32),
        pltpu.VMEM((80,), jnp.int32),
        pltpu.VMEM_SHARED((80, 128), jnp.float32),
    ],
)
def _deg_kernel(col_hbm, ew_hbm, out_hbm, cidx_all, wv_all, deg_loc,
                zbuf, idx80, deg_sh):
    c = lax.axis_index("c")
    s = lax.axis_index("s")
    wid = c * _NS + s
    z16 = jnp.zeros((16,), jnp.float32)
    i16 = lax.iota(jnp.int32, 16)

    pltpu.sync_copy(col_hbm.at[pl.ds(wid * _NCHUNK, _NCHUNK)], cidx_all)
    pltpu.sync_copy(ew_hbm.at[pl.ds(wid * _NCHUNK, _NCHUNK)], wv_all)

    for i in range(5):
        idx80[pl.ds(i * 16, 16)] = i16 + (i * 16)

    def zero_body(r, carry):
        for j in range(8):
            deg_loc[r, pl.ds(j * 16, 16)] = z16
        return carry

    lax.fori_loop(0, 80, zero_body, 0)

    def zb_body(r, carry):
        for j in range(8):
            zbuf[r, pl.ds(j * 16, 16)] = z16
        return carry

    lax.fori_loop(0, 16, zb_body, 0)

    @pl.when(s < 5)
    def _zero_sh():
        pltpu.sync_copy(zbuf, deg_sh.at[pl.ds(s * 16, 16)])

    del _zero_sh
    plsc.subcore_barrier()

    def chunk_body(ci, carry):
        def grp_body(g, carry2):
            cv = cidx_all[ci, pl.ds(g * 16, 16)]
            ev = wv_all[ci, pl.ds(g * 16, 16)]
            plsc.addupdate_scatter(
                deg_loc,
                [lax.shift_right_logical(cv, 7),
                 lax.bitwise_and(cv, 127)], ev)
            return carry2

        lax.fori_loop(0, _CH // 16, grp_body, 0)
        return carry

    lax.fori_loop(0, _NCHUNK, chunk_body, 0)

    pltpu.sync_copy(deg_loc, deg_sh.at[idx80], add=True)
    plsc.subcore_barrier()

    @pl.when(s < 5)
    def _writeout():
        pltpu.sync_copy(deg_sh.at[pl.ds(s * 16, 16)],
                        out_hbm.at[pl.ds(c * 80 + s * 16, 16)])

    del _writeout


_NBUF = 4
_CH0 = 64
_CH1 = 160 - _CH0


@functools.partial(
    pl.kernel,
    out_type=jax.ShapeDtypeStruct((_NC * _NP, _H), jnp.float32),
    mesh=_sc_mesh,
    compiler_params=pltpu.CompilerParams(needs_layout_passes=False, use_tc_tiling_on_sc=False),
    scratch_types=[
        pltpu.VMEM((_CH1, _CH), jnp.int32),
        pltpu.VMEM((_CH1, _CH), jnp.int32),
        pltpu.VMEM((_CH1, _CH), jnp.float32),
        pltpu.VMEM((_CH, _H), jnp.bfloat16),
        pltpu.VMEM((_CH, _H), jnp.bfloat16),
        pltpu.VMEM((_CH, _H), jnp.bfloat16),
        pltpu.VMEM((_CH, _H), jnp.bfloat16),
        pltpu.VMEM((_CH, _H), jnp.float32),
        pltpu.VMEM((_CH, _H), jnp.float32),
        pltpu.VMEM((_CH, _H), jnp.float32),
        pltpu.VMEM((_CH, _H), jnp.float32),
        pltpu.VMEM((_CH1,), jnp.int32),
        pltpu.VMEM_SHARED((_NP, _H), jnp.float32),
        pltpu.SemaphoreType.DMA,
        pltpu.SemaphoreType.DMA,
        pltpu.SemaphoreType.DMA,
        pltpu.SemaphoreType.DMA,
        pltpu.SemaphoreType.DMA,
        pltpu.SemaphoreType.DMA,
        pltpu.SemaphoreType.DMA,
        pltpu.SemaphoreType.DMA,
    ],
)
def _agg_kernel(y_hbm, row_hbm, col_hbm, ew_hbm, out_hbm,
                ridx_all, cidx_all, wv_all, rows0, rows1, rows2, rows3,
                frow0, frow1, frow2, frow3,
                ibuf, acc_sh,
                gs0, gs1, gs2, gs3, ss0, ss1, ss2, ss3):
    c = lax.axis_index("c")
    s = lax.axis_index("s")
    wid = c * _NS + s
    z16 = jnp.zeros((16,), jnp.float32)
    rowsb = [rows0, rows1, rows2, rows3]
    frowb = [frow0, frow1, frow2, frow3]
    gsem = [gs0, gs1, gs2, gs3]
    ssem = [ss0, ss1, ss2, ss3]

    nch = jnp.where(c == 0, _CH0, _CH1)
    cbase = jnp.where(c == 0, s * _CH0, _NS * _CH0 + s * _CH1)

    i16 = lax.iota(jnp.int32, 16)
    for i in range(_CH1 // 16):
        ibuf[pl.ds(i * 16, 16)] = jnp.minimum(cbase + i16 + i * 16,
                                              _NW * _NCHUNK - 1)
    pltpu.async_copy(row_hbm.at[ibuf], ridx_all, gs0)
    pltpu.async_copy(col_hbm.at[ibuf], cidx_all, gs1)
    pltpu.async_copy(ew_hbm.at[ibuf], wv_all, gs2)
    pltpu.make_async_copy(row_hbm.at[ibuf], ridx_all, gs0).wait()
    pltpu.make_async_copy(col_hbm.at[ibuf], cidx_all, gs1).wait()
    pltpu.make_async_copy(ew_hbm.at[ibuf], wv_all, gs2).wait()

    def zb_body(r, carry):
        for j in range(_H // 16):
            frow0[r, pl.ds(j * 16, 16)] = z16
        return carry

    lax.fori_loop(0, 128, zb_body, 0)
    for k in range(5):
        pltpu.sync_copy(frow0, acc_sh.at[pl.ds(s * _RPT + k * 128, 128)])
    plsc.subcore_barrier()

    splat_idx = [jnp.full((16,), l, jnp.int32) for l in range(16)]

    def gather_start(ci, b):
        pltpu.async_copy(y_hbm.at[ridx_all.at[ci]], rowsb[b], gsem[b])

    def gather_wait(ci, b):
        pltpu.make_async_copy(y_hbm.at[ridx_all.at[ci]], rowsb[b],
                              gsem[b]).wait()

    def scatter_start(ci, b):
        pltpu.async_copy(frowb[b], acc_sh.at[cidx_all.at[ci]], ssem[b],
                         add=True)

    def scatter_wait(ci, b):
        pltpu.make_async_copy(frowb[b], acc_sh.at[cidx_all.at[ci]],
                              ssem[b]).wait()

    def scale(ci, b):
        rbuf = rowsb[b]
        fbuf = frowb[b]

        @plsc.parallel_loop(0, _CH // 16)
        def grp_body(g):
            wg = wv_all[ci, pl.ds(g * 16, 16)]
            for lane in range(16):
                ws = wg.at[splat_idx[lane]].get(mode="promise_in_bounds")
                k = g * 16 + lane
                for j in range(_H // 32):
                    v = rbuf[k, pl.ds(j * 32, 32)]
                    lo, hi = plsc.unpack(
                        v, format=plsc.PackFormat.INTERLEAVED,
                        preferred_element_type=jnp.float32)
                    fbuf[k, pl.ds(j * 32, 16)] = lo * ws
                    fbuf[k, pl.ds(j * 32 + 16, 16)] = hi * ws

        del grp_body

    gather_start(0, 0)
    gather_start(1, 1)
    for ci in (0, 1):
        gather_start(ci + 2, ci + 2)
        gather_wait(ci, ci)
        scale(ci, ci)
        scatter_start(ci, ci)

    def steady_body(it, carry):
        m = 2 + it * 4
        for db in range(4):
            ci = m + db
            b = (2 + db) % 4
            pb = db
            scatter_wait(ci - 2, pb)
            gather_start(ci + 2, pb)
            gather_wait(ci, b)
            scale(ci, b)
            scatter_start(ci, b)
        return carry

    lax.fori_loop(0, (nch - 4) // 4, steady_body, 0)

    for dci in (2, 1):
        ci = nch - dci
        b = 4 - dci
        gather_wait(ci, b)
        scale(ci, b)
        scatter_start(ci, b)
    for k in range(4):
        scatter_wait(nch - 4 + k, k)

    plsc.subcore_barrier()
    pltpu.sync_copy(acc_sh.at[pl.ds(s * _RPT, _RPT)],
                    out_hbm.at[pl.ds(c * _NP + s * _RPT, _RPT)])


def _prep_body(x_ref, w1t_ref, degp_ref, py_ref, xw_ref, y_ref, dis_ref, dinv_ref):
    deg = jnp.sum(degp_ref[0], axis=0) + 1.0
    dis = lax.rsqrt(deg)
    dinv = 1.0 / deg
    xw = jnp.dot(x_ref[...], w1t_ref[...], preferred_element_type=jnp.float32)
    xw_ref[...] = xw
    y_ref[...] = jnp.dot(xw * dis[:, None], py_ref[...],
                         preferred_element_type=jnp.float32
                         ).astype(jnp.bfloat16)
    dis_ref[0, 0, :] = dis
    dinv_ref[0, 0, :] = dinv


def _prep_call(x, w1t, degp, py):
    return pl.pallas_call(
        _prep_body,
        grid=(_NB,),
        in_specs=[
            pl.BlockSpec((_BM, _D_IN), lambda i: (i, 0)),
            pl.BlockSpec((_D_IN, _H), lambda i: (0, 0)),
            pl.BlockSpec((1, _NC, _BM), lambda i: (i, 0, 0)),
            pl.BlockSpec((_H, _H), lambda i: (0, 0)),
        ],
        out_specs=[
            pl.BlockSpec((_BM, _H), lambda i: (i, 0)),
            pl.BlockSpec((_BM, _H), lambda i: (i, 0)),
            pl.BlockSpec((1, 1, _BM), lambda i: (i, 0, 0)),
            pl.BlockSpec((1, 1, _BM), lambda i: (i, 0, 0)),
        ],
        out_shape=[
            jax.ShapeDtypeStruct((_N, _H), jnp.float32),
            jax.ShapeDtypeStruct((_N, _H), jnp.bfloat16),
            jax.ShapeDtypeStruct((_NB, 1, _BM), jnp.float32),
            jax.ShapeDtypeStruct((_NB, 1, _BM), jnp.float32),
        ],
    )(x, w1t, degp, py)


def _mid_body(accp_ref, xw_ref, dis_ref, dinv_ref, s1_ref, t1_ref, w2t_ref,
              py_ref, xw2_ref, y2_ref):
    acc = accp_ref[0] + accp_ref[1]
    dis = dis_ref[0, 0, :]
    out = dis[:, None] * acc + dinv_ref[0, 0, :][:, None] * xw_ref[...]
    h = jnp.maximum(out * s1_ref[...][None, :] + t1_ref[...][None, :], 0.0)
    xw2 = jnp.dot(h, w2t_ref[...], preferred_element_type=jnp.float32)
    xw2_ref[...] = xw2
    y2_ref[...] = jnp.dot(xw2 * dis[:, None], py_ref[...],
                          preferred_element_type=jnp.float32
                          ).astype(jnp.bfloat16)


def _mid_call(accp, xw, dis, dinv, s1, t1, w2t, py):
    return pl.pallas_call(
        _mid_body,
        grid=(_NB,),
        in_specs=[
            pl.BlockSpec((2, _BM, _H), lambda i: (0, i, 0)),
            pl.BlockSpec((_BM, _H), lambda i: (i, 0)),
            pl.BlockSpec((1, 1, _BM), lambda i: (i, 0, 0)),
            pl.BlockSpec((1, 1, _BM), lambda i: (i, 0, 0)),
            pl.BlockSpec((_H,), lambda i: (0,)),
            pl.BlockSpec((_H,), lambda i: (0,)),
            pl.BlockSpec((_H, _H), lambda i: (0, 0)),
            pl.BlockSpec((_H, _H), lambda i: (0, 0)),
        ],
        out_specs=[
            pl.BlockSpec((_BM, _H), lambda i: (i, 0)),
            pl.BlockSpec((_BM, _H), lambda i: (i, 0)),
        ],
        out_shape=[
            jax.ShapeDtypeStruct((_N, _H), jnp.float32),
            jax.ShapeDtypeStruct((_N, _H), jnp.bfloat16),
        ],
    )(accp, xw, dis, dinv, s1, t1, w2t, py)


def _final_body(accp_ref, xw2_ref, dis_ref, dinv_ref, s2_ref, t2_ref,
                batch_ref, l1t_ref, l1b_ref, l2t_ref, l2b_ref,
                out_ref, psum_ref, cnt_ref):
    i = pl.program_id(0)

    @pl.when(i == 0)
    def _init():
        psum_ref[...] = jnp.zeros((_G, _H), jnp.float32)
        cnt_ref[...] = jnp.zeros((_G,), jnp.float32)

    acc = accp_ref[0] + accp_ref[1]
    out = (dis_ref[0, 0, :][:, None] * acc
           + dinv_ref[0, 0, :][:, None] * xw2_ref[...])
    h = jnp.maximum(out * s2_ref[...][None, :] + t2_ref[...][None, :], 0.0)

    b = batch_ref[0, 0, :]
    oh = (b[:, None] == lax.broadcasted_iota(jnp.int32, (1, _G), 1)
          ).astype(jnp.float32)
    psum_ref[...] += lax.dot_general(oh, h, (((0,), (0,)), ((), ())),
                                     preferred_element_type=jnp.float32)
    cnt_ref[...] += jnp.sum(oh, axis=0)

    @pl.when(i == _NB - 1)
    def _fin():
        g = psum_ref[...] / jnp.maximum(cnt_ref[...], 1.0)[:, None]
        z = jnp.maximum(
            jnp.dot(g, l1t_ref[...], preferred_element_type=jnp.float32)
            + l1b_ref[...][None, :], 0.0)
        o = jnp.dot(z, l2t_ref[...], preferred_element_type=jnp.float32)
        out_ref[...] = o[:, 0] + l2b_ref[...]

    del _init, _fin


def _final_call(accp, xw2, dis, dinv, s2, t2, batch3d, l1t, l1b, l2t, l2b):
    return pl.pallas_call(
        _final_body,
        grid=(_NB,),
        in_specs=[
            pl.BlockSpec((2, _BM, _H), lambda i: (0, i, 0)),
            pl.BlockSpec((_BM, _H), lambda i: (i, 0)),
            pl.BlockSpec((1, 1, _BM), lambda i: (i, 0, 0)),
            pl.BlockSpec((1, 1, _BM), lambda i: (i, 0, 0)),
            pl.BlockSpec((_H,), lambda i: (0,)),
            pl.BlockSpec((_H,), lambda i: (0,)),
            pl.BlockSpec((1, 1, _BM), lambda i: (i, 0, 0)),
            pl.BlockSpec((_H, _H // 2), lambda i: (0, 0)),
            pl.BlockSpec((_H // 2,), lambda i: (0,)),
            pl.BlockSpec((_H // 2, 1), lambda i: (0, 0)),
            pl.BlockSpec((1,), lambda i: (0,)),
        ],
        out_specs=pl.BlockSpec((_G,), lambda i: (0,)),
        out_shape=jax.ShapeDtypeStruct((_G,), jnp.float32),
        scratch_shapes=[
            pltpu.VMEM((_G, _H), jnp.float32),
            pltpu.VMEM((_G,), jnp.float32),
        ],
    )(accp, xw2, dis, dinv, s2, t2, batch3d, l1t, l1b, l2t, l2b)


def kernel(x, edge_index, edge_weight, batch, W1, b1, W2, b2,
           bn1_gamma, bn1_beta, bn1_mean, bn1_var,
           bn2_gamma, bn2_beta, bn2_mean, bn2_var,
           lin1_W, lin1_b, lin2_W, lin2_b):
    npad = _EPAD - _E
    row = jnp.concatenate(
        [edge_index[0].astype(jnp.int32), jnp.zeros((npad,), jnp.int32)])
    col = jnp.concatenate(
        [edge_index[1].astype(jnp.int32), jnp.full((npad,), _N, jnp.int32)])
    ew = jnp.concatenate(
        [edge_weight.astype(jnp.float32), jnp.zeros((npad,), jnp.float32)])
    batch3d = batch.astype(jnp.int32).reshape(_NB, 1, _BM)

    w1t = W1.T
    w2t = W2.T
    l1t = lin1_W.T
    l2t = lin2_W.T

    s1 = bn1_gamma * lax.rsqrt(bn1_var + _EPS)
    t1 = bn1_beta - bn1_mean * s1 + b1 * s1
    s2 = bn2_gamma * lax.rsqrt(bn2_var + _EPS)
    t2 = bn2_beta - bn2_mean * s2 + b2 * s2

    row2d = row.reshape(_NW * _NCHUNK, _CH)
    col2d = col.reshape(_NW * _NCHUNK, _CH)
    ew2d = ew.reshape(_NW * _NCHUNK, _CH)

    py = jnp.asarray(_PY)
    degp = (_deg_kernel(col2d, ew2d).reshape(_NC, _NP)[:, :_N]
            .reshape(_NC, _NB, _BM).transpose(1, 0, 2))
    xw1, y1, dis, dinv = _prep_call(x, w1t, degp, py)

    accp1 = _agg_kernel(y1, row2d, col2d, ew2d).reshape(_NC, _NP, _H)
    xw2, y2 = _mid_call(accp1, xw1, dis, dinv, s1, t1, w2t, py)

    accp2 = _agg_kernel(y2, row2d, col2d, ew2d).reshape(_NC, _NP, _H)
    out = _final_call(accp2, xw2, dis, dinv, s2, t2, batch3d,
                      l1t, lin1_b, l2t, lin2_b)
    return out


# speedup vs baseline: 1.0795x; 1.0795x over previous
import functools

import jax
import jax.numpy as jnp
from jax import lax
from jax.experimental import pallas as pl
from jax.experimental.pallas import tpu as pltpu
from jax.experimental.pallas import tpu_sc as plsc

_N = 10000
_E = 320000
_D_IN = 128
_H = 64
_G = 64
_EPS = 1e-5

_NC = 2
_NS = 16
_NW = _NC * _NS
_CH = 128
_NCHUNK = 80
_EPW = _CH * _NCHUNK
_EPAD = _NW * _EPW
_NP = 10240
_RPT = _NP // _NS

_BM = 2000
_NB = _N // _BM


def _interleave_perm_matrix():
    import numpy as _np
    perm = _np.zeros((_H,), _np.int64)
    for blk in range(_H // 32):
        for i in range(16):
            perm[blk * 32 + 2 * i] = blk * 32 + i
            perm[blk * 32 + 2 * i + 1] = blk * 32 + 16 + i
    m = _np.zeros((_H, _H), _np.float32)
    for col in range(_H):
        m[perm[col], col] = 1.0
    return m


_PY = _interleave_perm_matrix()

_sc_mesh = plsc.VectorSubcoreMesh(
    core_axis_name="c", subcore_axis_name="s", num_cores=_NC, num_subcores=_NS)


@functools.partial(
    pl.kernel,
    out_type=jax.ShapeDtypeStruct((_NC * 80, 128), jnp.float32),
    mesh=_sc_mesh,
    compiler_params=pltpu.CompilerParams(needs_layout_passes=False, use_tc_tiling_on_sc=False),
    scratch_types=[
        pltpu.VMEM((_NCHUNK, _CH), jnp.int32),
        pltpu.VMEM((_NCHUNK, _CH), jnp.float32),
        pltpu.VMEM((80, 128), jnp.float32),
        pltpu.VMEM((16, 128), jnp.float32),
        pltpu.VMEM((80,), jnp.int32),
        pltpu.VMEM_SHARED((80, 128), jnp.float32),
    ],
)
def _deg_kernel(col_hbm, ew_hbm, out_hbm, cidx_all, wv_all, deg_loc,
                zbuf, idx80, deg_sh):
    c = lax.axis_index("c")
    s = lax.axis_index("s")
    wid = c * _NS + s
    z16 = jnp.zeros((16,), jnp.float32)
    i16 = lax.iota(jnp.int32, 16)

    pltpu.sync_copy(col_hbm.at[pl.ds(wid * _NCHUNK, _NCHUNK)], cidx_all)
    pltpu.sync_copy(ew_hbm.at[pl.ds(wid * _NCHUNK, _NCHUNK)], wv_all)

    for i in range(5):
        idx80[pl.ds(i * 16, 16)] = i16 + (i * 16)

    def zero_body(r, carry):
        for j in range(8):
            deg_loc[r, pl.ds(j * 16, 16)] = z16
        return carry

    lax.fori_loop(0, 80, zero_body, 0)

    def zb_body(r, carry):
        for j in range(8):
            zbuf[r, pl.ds(j * 16, 16)] = z16
        return carry

    lax.fori_loop(0, 16, zb_body, 0)

    @pl.when(s < 5)
    def _zero_sh():
        pltpu.sync_copy(zbuf, deg_sh.at[pl.ds(s * 16, 16)])

    del _zero_sh
    plsc.subcore_barrier()

    def chunk_body(ci, carry):
        def grp_body(g, carry2):
            cv = cidx_all[ci, pl.ds(g * 16, 16)]
            ev = wv_all[ci, pl.ds(g * 16, 16)]
            plsc.addupdate_scatter(
                deg_loc,
                [lax.shift_right_logical(cv, 7),
                 lax.bitwise_and(cv, 127)], ev)
            return carry2

        lax.fori_loop(0, _CH // 16, grp_body, 0)
        return carry

    lax.fori_loop(0, _NCHUNK, chunk_body, 0)

    pltpu.sync_copy(deg_loc, deg_sh.at[idx80], add=True)
    plsc.subcore_barrier()

    @pl.when(s < 5)
    def _writeout():
        pltpu.sync_copy(deg_sh.at[pl.ds(s * 16, 16)],
                        out_hbm.at[pl.ds(c * 80 + s * 16, 16)])

    del _writeout


_NBUF = 4
_CH0 = 96
_CH1 = 160 - _CH0
_CHMAX = max(_CH0, _CH1)


@functools.partial(
    pl.kernel,
    out_type=jax.ShapeDtypeStruct((_NC * _NP, _H), jnp.float32),
    mesh=_sc_mesh,
    compiler_params=pltpu.CompilerParams(needs_layout_passes=False, use_tc_tiling_on_sc=False),
    scratch_types=[
        pltpu.VMEM((_CHMAX, _CH), jnp.int32),
        pltpu.VMEM((_CHMAX, _CH), jnp.int32),
        pltpu.VMEM((_CHMAX, _CH), jnp.float32),
        pltpu.VMEM((_CH, _H), jnp.bfloat16),
        pltpu.VMEM((_CH, _H), jnp.bfloat16),
        pltpu.VMEM((_CH, _H), jnp.bfloat16),
        pltpu.VMEM((_CH, _H), jnp.bfloat16),
        pltpu.VMEM((_CH, _H), jnp.float32),
        pltpu.VMEM((_CH, _H), jnp.float32),
        pltpu.VMEM((_CH, _H), jnp.float32),
        pltpu.VMEM((_CH, _H), jnp.float32),
        pltpu.VMEM((_CHMAX,), jnp.int32),
        pltpu.VMEM_SHARED((_NP, _H), jnp.float32),
        pltpu.SemaphoreType.DMA,
        pltpu.SemaphoreType.DMA,
        pltpu.SemaphoreType.DMA,
        pltpu.SemaphoreType.DMA,
        pltpu.SemaphoreType.DMA,
        pltpu.SemaphoreType.DMA,
        pltpu.SemaphoreType.DMA,
        pltpu.SemaphoreType.DMA,
    ],
)
def _agg_kernel(y_hbm, row_hbm, col_hbm, ew_hbm, out_hbm,
                ridx_all, cidx_all, wv_all, rows0, rows1, rows2, rows3,
                frow0, frow1, frow2, frow3,
                ibuf, acc_sh,
                gs0, gs1, gs2, gs3, ss0, ss1, ss2, ss3):
    c = lax.axis_index("c")
    s = lax.axis_index("s")
    wid = c * _NS + s
    z16 = jnp.zeros((16,), jnp.float32)
    rowsb = [rows0, rows1, rows2, rows3]
    frowb = [frow0, frow1, frow2, frow3]
    gsem = [gs0, gs1, gs2, gs3]
    ssem = [ss0, ss1, ss2, ss3]

    nch = jnp.where(c == 0, _CH0, _CH1)
    cbase = jnp.where(c == 0, s * _CH0, _NS * _CH0 + s * _CH1)

    i16 = lax.iota(jnp.int32, 16)
    for i in range(_CHMAX // 16):
        ibuf[pl.ds(i * 16, 16)] = jnp.minimum(cbase + i16 + i * 16,
                                              _NW * _NCHUNK - 1)
    pltpu.async_copy(row_hbm.at[ibuf], ridx_all, gs0)
    pltpu.async_copy(col_hbm.at[ibuf], cidx_all, gs1)
    pltpu.async_copy(ew_hbm.at[ibuf], wv_all, gs2)
    pltpu.make_async_copy(row_hbm.at[ibuf], ridx_all, gs0).wait()
    pltpu.make_async_copy(col_hbm.at[ibuf], cidx_all, gs1).wait()
    pltpu.make_async_copy(ew_hbm.at[ibuf], wv_all, gs2).wait()

    def zb_body(r, carry):
        for j in range(_H // 16):
            frow0[r, pl.ds(j * 16, 16)] = z16
        return carry

    lax.fori_loop(0, 128, zb_body, 0)
    for k in range(5):
        pltpu.sync_copy(frow0, acc_sh.at[pl.ds(s * _RPT + k * 128, 128)])
    plsc.subcore_barrier()

    splat_idx = [jnp.full((16,), l, jnp.int32) for l in range(16)]

    def gather_start(ci, b):
        pltpu.async_copy(y_hbm.at[ridx_all.at[ci]], rowsb[b], gsem[b])

    def gather_wait(ci, b):
        pltpu.make_async_copy(y_hbm.at[ridx_all.at[ci]], rowsb[b],
                              gsem[b]).wait()

    def scatter_start(ci, b):
        pltpu.async_copy(frowb[b], acc_sh.at[cidx_all.at[ci]], ssem[b],
                         add=True)

    def scatter_wait(ci, b):
        pltpu.make_async_copy(frowb[b], acc_sh.at[cidx_all.at[ci]],
                              ssem[b]).wait()

    def scale(ci, b):
        rbuf = rowsb[b]
        fbuf = frowb[b]

        @plsc.parallel_loop(0, _CH // 16)
        def grp_body(g):
            wg = wv_all[ci, pl.ds(g * 16, 16)]
            for lane in range(16):
                ws = wg.at[splat_idx[lane]].get(mode="promise_in_bounds")
                k = g * 16 + lane
                for j in range(_H // 32):
                    v = rbuf[k, pl.ds(j * 32, 32)]
                    lo, hi = plsc.unpack(
                        v, format=plsc.PackFormat.INTERLEAVED,
                        preferred_element_type=jnp.float32)
                    fbuf[k, pl.ds(j * 32, 16)] = lo * ws
                    fbuf[k, pl.ds(j * 32 + 16, 16)] = hi * ws

        del grp_body

    gather_start(0, 0)
    gather_start(1, 1)
    for ci in (0, 1):
        gather_start(ci + 2, ci + 2)
        gather_wait(ci, ci)
        scale(ci, ci)
        scatter_start(ci, ci)

    def steady_body(it, carry):
        m = 2 + it * 4
        for db in range(4):
            ci = m + db
            b = (2 + db) % 4
            pb = db
            scatter_wait(ci - 2, pb)
            gather_start(ci + 2, pb)
            gather_wait(ci, b)
            scale(ci, b)
            scatter_start(ci, b)
        return carry

    lax.fori_loop(0, (nch - 4) // 4, steady_body, 0)

    for dci in (2, 1):
        ci = nch - dci
        b = 4 - dci
        gather_wait(ci, b)
        scale(ci, b)
        scatter_start(ci, b)
    for k in range(4):
        scatter_wait(nch - 4 + k, k)

    plsc.subcore_barrier()
    pltpu.sync_copy(acc_sh.at[pl.ds(s * _RPT, _RPT)],
                    out_hbm.at[pl.ds(c * _NP + s * _RPT, _RPT)])


def _prep_body(x_ref, w1t_ref, degp_ref, py_ref, xw_ref, y_ref, dis_ref, dinv_ref):
    deg = jnp.sum(degp_ref[0], axis=0) + 1.0
    dis = lax.rsqrt(deg)
    dinv = 1.0 / deg
    xw = jnp.dot(x_ref[...], w1t_ref[...], preferred_element_type=jnp.float32)
    xw_ref[...] = xw
    y_ref[...] = jnp.dot(xw * dis[:, None], py_ref[...],
                         preferred_element_type=jnp.float32
                         ).astype(jnp.bfloat16)
    dis_ref[0, 0, :] = dis
    dinv_ref[0, 0, :] = dinv


def _prep_call(x, w1t, degp, py):
    return pl.pallas_call(
        _prep_body,
        grid=(_NB,),
        in_specs=[
            pl.BlockSpec((_BM, _D_IN), lambda i: (i, 0)),
            pl.BlockSpec((_D_IN, _H), lambda i: (0, 0)),
            pl.BlockSpec((1, _NC, _BM), lambda i: (i, 0, 0)),
            pl.BlockSpec((_H, _H), lambda i: (0, 0)),
        ],
        out_specs=[
            pl.BlockSpec((_BM, _H), lambda i: (i, 0)),
            pl.BlockSpec((_BM, _H), lambda i: (i, 0)),
            pl.BlockSpec((1, 1, _BM), lambda i: (i, 0, 0)),
            pl.BlockSpec((1, 1, _BM), lambda i: (i, 0, 0)),
        ],
        out_shape=[
            jax.ShapeDtypeStruct((_N, _H), jnp.float32),
            jax.ShapeDtypeStruct((_N, _H), jnp.bfloat16),
            jax.ShapeDtypeStruct((_NB, 1, _BM), jnp.float32),
            jax.ShapeDtypeStruct((_NB, 1, _BM), jnp.float32),
        ],
    )(x, w1t, degp, py)


def _mid_body(accp_ref, xw_ref, dis_ref, dinv_ref, s1_ref, t1_ref, w2t_ref,
              py_ref, xw2_ref, y2_ref):
    acc = accp_ref[0] + accp_ref[1]
    dis = dis_ref[0, 0, :]
    out = dis[:, None] * acc + dinv_ref[0, 0, :][:, None] * xw_ref[...]
    h = jnp.maximum(out * s1_ref[...][None, :] + t1_ref[...][None, :], 0.0)
    xw2 = jnp.dot(h, w2t_ref[...], preferred_element_type=jnp.float32)
    xw2_ref[...] = xw2
    y2_ref[...] = jnp.dot(xw2 * dis[:, None], py_ref[...],
                          preferred_element_type=jnp.float32
                          ).astype(jnp.bfloat16)


def _mid_call(accp, xw, dis, dinv, s1, t1, w2t, py):
    return pl.pallas_call(
        _mid_body,
        grid=(_NB,),
        in_specs=[
            pl.BlockSpec((2, _BM, _H), lambda i: (0, i, 0)),
            pl.BlockSpec((_BM, _H), lambda i: (i, 0)),
            pl.BlockSpec((1, 1, _BM), lambda i: (i, 0, 0)),
            pl.BlockSpec((1, 1, _BM), lambda i: (i, 0, 0)),
            pl.BlockSpec((_H,), lambda i: (0,)),
            pl.BlockSpec((_H,), lambda i: (0,)),
            pl.BlockSpec((_H, _H), lambda i: (0, 0)),
            pl.BlockSpec((_H, _H), lambda i: (0, 0)),
        ],
        out_specs=[
            pl.BlockSpec((_BM, _H), lambda i: (i, 0)),
            pl.BlockSpec((_BM, _H), lambda i: (i, 0)),
        ],
        out_shape=[
            jax.ShapeDtypeStruct((_N, _H), jnp.float32),
            jax.ShapeDtypeStruct((_N, _H), jnp.bfloat16),
        ],
    )(accp, xw, dis, dinv, s1, t1, w2t, py)


def _final_body(accp_ref, xw2_ref, dis_ref, dinv_ref, s2_ref, t2_ref,
                batch_ref, l1t_ref, l1b_ref, l2t_ref, l2b_ref,
                out_ref, psum_ref, cnt_ref):
    i = pl.program_id(0)

    @pl.when(i == 0)
    def _init():
        psum_ref[...] = jnp.zeros((_G, _H), jnp.float32)
        cnt_ref[...] = jnp.zeros((_G,), jnp.float32)

    acc = accp_ref[0] + accp_ref[1]
    out = (dis_ref[0, 0, :][:, None] * acc
           + dinv_ref[0, 0, :][:, None] * xw2_ref[...])
    h = jnp.maximum(out * s2_ref[...][None, :] + t2_ref[...][None, :], 0.0)

    b = batch_ref[0, 0, :]
    oh = (b[:, None] == lax.broadcasted_iota(jnp.int32, (1, _G), 1)
          ).astype(jnp.float32)
    psum_ref[...] += lax.dot_general(oh, h, (((0,), (0,)), ((), ())),
                                     preferred_element_type=jnp.float32)
    cnt_ref[...] += jnp.sum(oh, axis=0)

    @pl.when(i == _NB - 1)
    def _fin():
        g = psum_ref[...] / jnp.maximum(cnt_ref[...], 1.0)[:, None]
        z = jnp.maximum(
            jnp.dot(g, l1t_ref[...], preferred_element_type=jnp.float32)
            + l1b_ref[...][None, :], 0.0)
        o = jnp.dot(z, l2t_ref[...], preferred_element_type=jnp.float32)
        out_ref[...] = o[:, 0] + l2b_ref[...]

    del _init, _fin


def _final_call(accp, xw2, dis, dinv, s2, t2, batch3d, l1t, l1b, l2t, l2b):
    return pl.pallas_call(
        _final_body,
        grid=(_NB,),
        in_specs=[
            pl.BlockSpec((2, _BM, _H), lambda i: (0, i, 0)),
            pl.BlockSpec((_BM, _H), lambda i: (i, 0)),
            pl.BlockSpec((1, 1, _BM), lambda i: (i, 0, 0)),
            pl.BlockSpec((1, 1, _BM), lambda i: (i, 0, 0)),
            pl.BlockSpec((_H,), lambda i: (0,)),
            pl.BlockSpec((_H,), lambda i: (0,)),
            pl.BlockSpec((1, 1, _BM), lambda i: (i, 0, 0)),
            pl.BlockSpec((_H, _H // 2), lambda i: (0, 0)),
            pl.BlockSpec((_H // 2,), lambda i: (0,)),
            pl.BlockSpec((_H // 2, 1), lambda i: (0, 0)),
            pl.BlockSpec((1,), lambda i: (0,)),
        ],
        out_specs=pl.BlockSpec((_G,), lambda i: (0,)),
        out_shape=jax.ShapeDtypeStruct((_G,), jnp.float32),
        scratch_shapes=[
            pltpu.VMEM((_G, _H), jnp.float32),
            pltpu.VMEM((_G,), jnp.float32),
        ],
    )(accp, xw2, dis, dinv, s2, t2, batch3d, l1t, l1b, l2t, l2b)


def kernel(x, edge_index, edge_weight, batch, W1, b1, W2, b2,
           bn1_gamma, bn1_beta, bn1_mean, bn1_var,
           bn2_gamma, bn2_beta, bn2_mean, bn2_var,
           lin1_W, lin1_b, lin2_W, lin2_b):
    npad = _EPAD - _E
    row = jnp.concatenate(
        [edge_index[0].astype(jnp.int32), jnp.zeros((npad,), jnp.int32)])
    col = jnp.concatenate(
        [edge_index[1].astype(jnp.int32), jnp.full((npad,), _N, jnp.int32)])
    ew = jnp.concatenate(
        [edge_weight.astype(jnp.float32), jnp.zeros((npad,), jnp.float32)])
    batch3d = batch.astype(jnp.int32).reshape(_NB, 1, _BM)

    w1t = W1.T
    w2t = W2.T
    l1t = lin1_W.T
    l2t = lin2_W.T

    s1 = bn1_gamma * lax.rsqrt(bn1_var + _EPS)
    t1 = bn1_beta - bn1_mean * s1 + b1 * s1
    s2 = bn2_gamma * lax.rsqrt(bn2_var + _EPS)
    t2 = bn2_beta - bn2_mean * s2 + b2 * s2

    row2d = row.reshape(_NW * _NCHUNK, _CH)
    col2d = col.reshape(_NW * _NCHUNK, _CH)
    ew2d = ew.reshape(_NW * _NCHUNK, _CH)

    py = jnp.asarray(_PY)
    degp = (_deg_kernel(col2d, ew2d).reshape(_NC, _NP)[:, :_N]
            .reshape(_NC, _NB, _BM).transpose(1, 0, 2))
    xw1, y1, dis, dinv = _prep_call(x, w1t, degp, py)

    accp1 = _agg_kernel(y1, row2d, col2d, ew2d).reshape(_NC, _NP, _H)
    xw2, y2 = _mid_call(accp1, xw1, dis, dinv, s1, t1, w2t, py)

    accp2 = _agg_kernel(y2, row2d, col2d, ew2d).reshape(_NC, _NP, _H)
    out = _final_call(accp2, xw2, dis, dinv, s2, t2, batch3d,
                      l1t, lin1_b, l2t, lin2_b)
    return out
